# Initial kernel scaffold; baseline (speedup 1.0000x reference)
import jax
import jax.numpy as jnp
from jax.experimental import pallas as pl


def kernel(x, edge_features, E_idx, W1, b1, W2, b2):
    raise NotImplementedError("write your pallas kernel here")


# speedup vs baseline: 12.4282x; 12.4282x over previous
import jax
import jax.numpy as jnp
from jax.experimental import pallas as pl
from jax.experimental.pallas import tpu as pltpu
from jax.experimental.pallas import tpu_sc as plsc

N, K, C, EDGE = 10000, 32, 128, 16
NUM_EDGES = N * K

GATHER_WINDOW = 256
NB = 400
EB = NB * K


def _sc_gather(x2d, idx_flat):
    mesh = plsc.VectorSubcoreMesh(core_axis_name="core", subcore_axis_name="subcore")
    idx2d = idx_flat.reshape(1, NUM_EDGES)

    @pl.kernel(
        out_type=jax.ShapeDtypeStruct((NUM_EDGES, C), jnp.float32),
        mesh=mesh,
    )
    def kern(x_hbm, i_hbm, o_hbm):
        def body(i_vmem, o_vmem):
            pltpu.sync_copy(x_hbm.at[i_vmem.at[0]], o_vmem)

        pltpu.emit_pipeline(
            body,
            grid=(NUM_EDGES // GATHER_WINDOW,),
            in_specs=[pl.BlockSpec((1, GATHER_WINDOW), lambda i: (0, i))],
            out_specs=[pl.BlockSpec((GATHER_WINDOW, C), lambda i: (i, 0))],
            core_axis_name=("core", "subcore"),
            dimension_semantics=(pltpu.PARALLEL,),
        )(i_hbm, o_hbm)

    return kern(x2d, idx2d)


def _filter_body(ef_ref, xj_ref, w1_ref, b1_ref, w2_ref, b2_ref, out_ref):
    s1 = jnp.dot(ef_ref[...], w1_ref[...], preferred_element_type=jnp.float32)
    s1 = s1 + b1_ref[...]
    u = s1 + s1 * jax.lax.erf(s1)
    s2 = jnp.dot(u, w2_ref[...], preferred_element_type=jnp.float32)
    s2 = s2 + b2_ref[...]
    v = s2 + s2 * jax.lax.erf(s2)
    prod = v * xj_ref[...]
    out_ref[...] = prod.reshape(NB, K, C).sum(axis=1)


def _tc_fused(ef2, xj, w1, b1, w2, b2):
    return pl.pallas_call(
        _filter_body,
        grid=(N // NB,),
        in_specs=[
            pl.BlockSpec((EB, EDGE), lambda i: (i, 0)),
            pl.BlockSpec((EB, C), lambda i: (i, 0)),
            pl.BlockSpec((EDGE, C), lambda i: (0, 0)),
            pl.BlockSpec((1, C), lambda i: (0, 0)),
            pl.BlockSpec((C, C), lambda i: (0, 0)),
            pl.BlockSpec((1, C), lambda i: (0, 0)),
        ],
        out_specs=pl.BlockSpec((NB, C), lambda i: (i, 0)),
        out_shape=jax.ShapeDtypeStruct((N, C), jnp.float32),
    )(ef2, xj, w1, b1, w2, b2)


_INV_SQRT2 = 0.7071067811865476


def kernel(x, edge_features, E_idx, W1, b1, W2, b2):
    x2d = x[0] * _INV_SQRT2
    idx_flat = E_idx[0].reshape(NUM_EDGES).astype(jnp.int32)
    ef2 = edge_features[0].reshape(NUM_EDGES, EDGE)
    w1s = W1 * _INV_SQRT2
    b1s = b1.reshape(1, C) * _INV_SQRT2
    w2s = W2 * 0.5
    b2s = b2.reshape(1, C) * _INV_SQRT2
    xj = _sc_gather(x2d, idx_flat)
    out = _tc_fused(ef2, xj, w1s, b1s, w2s, b2s)
    return out.reshape(1, N, C)


# speedup vs baseline: 15.4080x; 1.2398x over previous
import jax
import jax.numpy as jnp
from jax.experimental import pallas as pl
from jax.experimental.pallas import tpu as pltpu
from jax.experimental.pallas import tpu_sc as plsc

N, K, C, EDGE = 10000, 32, 128, 16
NUM_EDGES = N * K

GATHER_WINDOW = 256
NB = 400


def _sc_gather(x2d, idx_flat):
    mesh = plsc.VectorSubcoreMesh(core_axis_name="core", subcore_axis_name="subcore")
    idx2d = idx_flat.reshape(1, NUM_EDGES)

    @pl.kernel(
        out_type=jax.ShapeDtypeStruct((NUM_EDGES, C), jnp.float32),
        mesh=mesh,
    )
    def kern(x_hbm, i_hbm, o_hbm):
        def body(i_vmem, o_vmem):
            pltpu.sync_copy(x_hbm.at[i_vmem.at[0]], o_vmem)

        pltpu.emit_pipeline(
            body,
            grid=(NUM_EDGES // GATHER_WINDOW,),
            in_specs=[pl.BlockSpec((1, GATHER_WINDOW), lambda i: (0, i))],
            out_specs=[pl.BlockSpec((GATHER_WINDOW, C), lambda i: (i, 0))],
            core_axis_name=("core", "subcore"),
            dimension_semantics=(pltpu.PARALLEL,),
        )(i_hbm, o_hbm)

    return kern(x2d, idx2d)


def _filter_body(ef_ref, xj_ref, w1_ref, b1_ref, w2_ref, b2_ref, out_ref):
    efk = ef_ref[0]
    s1 = jax.lax.dot_general(
        efk, w1_ref[...], (((0,), (0,)), ((), ())),
        preferred_element_type=jnp.float32,
    )
    s1 = s1 + b1_ref[...]
    u = s1 + s1 * jax.lax.erf(s1)
    s2 = jnp.dot(u, w2_ref[...], preferred_element_type=jnp.float32)
    s2 = s2 + b2_ref[...]
    v = s2 + s2 * jax.lax.erf(s2)
    contrib = v * xj_ref[0]

    @pl.when(pl.program_id(0) == 0)
    def _():
        out_ref[...] = contrib

    @pl.when(pl.program_id(0) != 0)
    def _():
        out_ref[...] = out_ref[...] + contrib


def _tc_fused(ef_t, xj3, w1, b1, w2, b2):
    return pl.pallas_call(
        _filter_body,
        grid=(K,),
        in_specs=[
            pl.BlockSpec((1, EDGE, N), lambda k: (k, 0, 0)),
            pl.BlockSpec((1, N, C), lambda k: (k, 0, 0)),
            pl.BlockSpec((EDGE, C), lambda k: (0, 0)),
            pl.BlockSpec((1, C), lambda k: (0, 0)),
            pl.BlockSpec((C, C), lambda k: (0, 0)),
            pl.BlockSpec((1, C), lambda k: (0, 0)),
        ],
        out_specs=pl.BlockSpec((N, C), lambda k: (0, 0)),
        out_shape=jax.ShapeDtypeStruct((N, C), jnp.float32),
    )(ef_t, xj3, w1, b1, w2, b2)


_INV_SQRT2 = 0.7071067811865476


def kernel(x, edge_features, E_idx, W1, b1, W2, b2):
    x2d = x[0] * _INV_SQRT2
    idx_flat = jnp.transpose(E_idx[0], (1, 0)).reshape(NUM_EDGES).astype(jnp.int32)
    ef_t = jnp.transpose(edge_features[0], (1, 2, 0))
    w1s = W1 * _INV_SQRT2
    b1s = b1.reshape(1, C) * _INV_SQRT2
    w2s = W2 * 0.5
    b2s = b2.reshape(1, C) * _INV_SQRT2
    xj = _sc_gather(x2d, idx_flat)
    xj3 = xj.reshape(K, N, C)
    out = _tc_fused(ef_t, xj3, w1s, b1s, w2s, b2s)
    return out.reshape(1, N, C)
